# baseline (speedup 1.0000x reference)
import functools

import jax
import jax.numpy as jnp
from jax import lax
from jax.experimental import pallas as pl
from jax.experimental.pallas import tpu as pltpu
from jax.experimental.pallas import tpu_sc as plsc

N = 20000
NPAD = 20480
FROWS = 160
COLS = 128
NUM_CLASSES = 2
TOP_K = 750
CONF_THRESH = 0.75
NMS_THRESH = 0.3
NMS_TOP_K = 5000
VAR0 = 0.1
VAR1 = 0.2
ONE_BITS = 0x3F800000

NW = 32
CHUNK = NPAD // NW
CAP = 240
BUF = 256
CROWS = NW * BUF // COLS


def _nms_core(scores, idxm, lxv, lyv, lwv, lhv, pcxv, pcyv, pwv, phv,
              out_ref, cur_ref, x1r, y1r, x2r, y2r, ar):
    f32 = jnp.float32
    neg_inf = f32(-jnp.inf)

    cx = pcxv + lxv * VAR0 * pwv
    cy = pcyv + lyv * VAR0 * phv
    w = pwv * jnp.exp(lwv * VAR1)
    h = phv * jnp.exp(lhv * VAR1)
    x1 = cx - w / 2
    y1 = cy - h / 2
    x2 = x1 + w
    y2 = y1 + h
    x1r[...] = x1
    y1r[...] = y1
    x2r[...] = x2
    y2r[...] = y2
    ar[...] = (x2 - x1) * (y2 - y1)

    valid = scores > CONF_THRESH
    bits = jax.lax.bitcast_convert_type(scores, jnp.int32)

    def bs_body(_, carry):
        lo, hi = carry
        mid = lo + (hi - lo + 1) // 2
        cnt = jnp.sum(jnp.where(valid & (bits >= mid), 1, 0))
        take = cnt >= NMS_TOP_K
        return jnp.where(take, mid, lo), jnp.where(take, hi, mid - 1)

    thr, _ = lax.fori_loop(0, 32, bs_body, (jnp.int32(0), jnp.int32(ONE_BITS)))

    c_gt = jnp.sum(jnp.where(valid & (bits > thr), 1, 0))
    m_need = NMS_TOP_K - c_gt
    tie = valid & (bits == thr)

    def bs2_body(_, carry):
        lo, hi = carry
        mid = lo + (hi - lo + 1) // 2
        cnt = jnp.sum(jnp.where(tie & (idxm >= mid), 1, 0))
        take = cnt >= m_need
        return jnp.where(take, mid, lo), jnp.where(take, hi, mid - 1)

    cstar, _ = lax.fori_loop(0, 16, bs2_body, (jnp.int32(0), jnp.int32(NPAD)))

    active = valid & ((bits > thr) | (tie & (idxm >= cstar)))
    cur_ref[...] = jnp.where(active, scores, neg_inf)
    out_ref[...] = jnp.zeros((TOP_K, 5), f32)

    def cond(carry):
        t, alive = carry
        return (t < TOP_K) & alive

    def body(carry):
        t, _ = carry
        cur = cur_ref[...]
        mval = jnp.max(cur)
        has = mval > neg_inf

        @pl.when(has)
        def _():
            i = jnp.max(jnp.where(cur == mval, idxm, -1))
            sel = idxm == i
            x1v = x1r[...]
            y1v = y1r[...]
            x2v = x2r[...]
            y2v = y2r[...]
            av = ar[...]
            zero = f32(0.0)
            x1i = jnp.sum(jnp.where(sel, x1v, zero))
            y1i = jnp.sum(jnp.where(sel, y1v, zero))
            x2i = jnp.sum(jnp.where(sel, x2v, zero))
            y2i = jnp.sum(jnp.where(sel, y2v, zero))
            ai = jnp.sum(jnp.where(sel, av, zero))
            xx1 = jnp.maximum(x1v, x1i)
            yy1 = jnp.maximum(y1v, y1i)
            xx2 = jnp.minimum(x2v, x2i)
            yy2 = jnp.minimum(y2v, y2i)
            iw = jnp.maximum(xx2 - xx1, zero)
            ih = jnp.maximum(yy2 - yy1, zero)
            inter = iw * ih
            union = av - inter + ai
            iou = inter / union
            supp = (iou > NMS_THRESH) | sel
            cur_ref[...] = jnp.where(supp, neg_inf, cur)
            out_ref[pl.ds(t, 1), 0:1] = jnp.full((1, 1), mval, f32)
            out_ref[pl.ds(t, 1), 1:2] = jnp.full((1, 1), x1i, f32)
            out_ref[pl.ds(t, 1), 2:3] = jnp.full((1, 1), y1i, f32)
            out_ref[pl.ds(t, 1), 3:4] = jnp.full((1, 1), x2i, f32)
            out_ref[pl.ds(t, 1), 4:5] = jnp.full((1, 1), y2i, f32)

        return t + 1, has

    lax.while_loop(cond, body, (jnp.int32(0), jnp.bool_(True)))


def _nms_full_kernel(lx, ly, lw, lh, pcx, pcy, pw, ph, sc,
                     out_ref, cur_ref, x1r, y1r, x2r, y2r, ar):
    ridx = lax.broadcasted_iota(jnp.int32, (FROWS, COLS), 0)
    cidx = lax.broadcasted_iota(jnp.int32, (FROWS, COLS), 1)
    idxm = ridx * COLS + cidx
    _nms_core(sc[...], idxm, lx[...], ly[...], lw[...], lh[...],
              pcx[...], pcy[...], pw[...], ph[...],
              out_ref, cur_ref, x1r, y1r, x2r, y2r, ar)


def _nms_compact_kernel(sc, idx, lx, ly, lw, lh, pcx, pcy, pw, ph, out_ref):
    f32 = jnp.float32
    neg_inf = f32(-jnp.inf)
    scores = sc[...]
    idxm = idx[...]

    cx = pcx[...] + lx[...] * VAR0 * pw[...]
    cy = pcy[...] + ly[...] * VAR0 * ph[...]
    w = pw[...] * jnp.exp(lw[...] * VAR1)
    h = ph[...] * jnp.exp(lh[...] * VAR1)
    x1 = cx - w / 2
    y1 = cy - h / 2
    x2 = x1 + w
    y2 = y1 + h
    area = (x2 - x1) * (y2 - y1)

    valid = scores > CONF_THRESH
    bits = jax.lax.bitcast_convert_type(scores, jnp.int32)

    def bs_body(_, carry):
        lo, hi = carry
        mid = lo + (hi - lo + 1) // 2
        cnt = jnp.sum(jnp.where(valid & (bits >= mid), 1, 0))
        take = cnt >= NMS_TOP_K
        return jnp.where(take, mid, lo), jnp.where(take, hi, mid - 1)

    thr, _ = lax.fori_loop(0, 32, bs_body, (jnp.int32(0), jnp.int32(ONE_BITS)))
    c_gt = jnp.sum(jnp.where(valid & (bits > thr), 1, 0))
    m_need = NMS_TOP_K - c_gt
    tie = valid & (bits == thr)

    def bs2_body(_, carry):
        lo, hi = carry
        mid = lo + (hi - lo + 1) // 2
        cnt = jnp.sum(jnp.where(tie & (idxm >= mid), 1, 0))
        take = cnt >= m_need
        return jnp.where(take, mid, lo), jnp.where(take, hi, mid - 1)

    cstar, _ = lax.fori_loop(0, 16, bs2_body, (jnp.int32(0), jnp.int32(NPAD)))

    active = valid & ((bits > thr) | (tie & (idxm >= cstar)))
    cur0 = jnp.where(active, scores, neg_inf)
    out_ref[...] = jnp.zeros((TOP_K, 5), f32)

    def cond(carry):
        t, alive, _ = carry
        return (t < TOP_K) & alive

    def body(carry):
        t, _, cur = carry
        zero = f32(0.0)
        mx, sec = cur, jnp.full_like(cur, neg_inf)
        r = CROWS
        while r > 1:
            r //= 2
            ma, mb = mx[:r], mx[r:]
            sa, sb = sec[:r], sec[r:]
            mx = jnp.maximum(ma, mb)
            sec = jnp.maximum(jnp.minimum(ma, mb), jnp.maximum(sa, sb))
        c = COLS
        while c > 1:
            c //= 2
            ma, mb = mx[:, :c], mx[:, c:]
            sa, sb = sec[:, :c], sec[:, c:]
            mx = jnp.maximum(ma, mb)
            sec = jnp.maximum(jnp.minimum(ma, mb), jnp.maximum(sa, sb))
        mval = mx[0, 0]
        m2 = sec[0, 0]
        has = mval > neg_inf
        has2 = m2 > neg_inf
        sel1m = cur == mval
        sel2m = (cur == m2) & (~sel1m)
        i1 = jnp.max(jnp.where(sel1m, idxm, -1))
        n1 = jnp.sum(jnp.where(sel1m, 1, 0))
        i2 = jnp.max(jnp.where(sel2m, idxm, -1))
        n2 = jnp.sum(jnp.where(sel2m, 1, 0))

        sel1 = idxm == i1
        x1a = jnp.sum(jnp.where(sel1, x1, zero))
        y1a = jnp.sum(jnp.where(sel1, y1, zero))
        x2a = jnp.sum(jnp.where(sel1, x2, zero))
        y2a = jnp.sum(jnp.where(sel1, y2, zero))
        aa = (x2a - x1a) * (y2a - y1a)
        iwa = jnp.maximum(jnp.minimum(x2, x2a) - jnp.maximum(x1, x1a), zero)
        iha = jnp.maximum(jnp.minimum(y2, y2a) - jnp.maximum(y1, y1a), zero)
        intera = iwa * iha
        ioua = intera / (area - intera + aa)

        sel2 = idxm == i2
        x1b = jnp.sum(jnp.where(sel2, x1, zero))
        y1b = jnp.sum(jnp.where(sel2, y1, zero))
        x2b = jnp.sum(jnp.where(sel2, x2, zero))
        y2b = jnp.sum(jnp.where(sel2, y2, zero))
        ab = (x2b - x1b) * (y2b - y1b)
        iwb = jnp.maximum(jnp.minimum(x2, x2b) - jnp.maximum(x1, x1b), zero)
        ihb = jnp.maximum(jnp.minimum(y2, y2b) - jnp.maximum(y1, y1b), zero)
        interb = iwb * ihb
        ioub = interb / (area - interb + ab)

        iw12 = jnp.maximum(jnp.minimum(x2b, x2a) - jnp.maximum(x1b, x1a), zero)
        ih12 = jnp.maximum(jnp.minimum(y2b, y2a) - jnp.maximum(y1b, y1a), zero)
        inter12 = iw12 * ih12
        iou12 = inter12 / (ab - inter12 + aa)

        spec = (has2 & (n1 == 1) & (n2 == 1)
                & (iou12 <= NMS_THRESH) & (t < TOP_K - 1))

        cur = jnp.where(has & (ioua > NMS_THRESH), neg_inf, cur)
        cur = jnp.where(spec & (ioub > NMS_THRESH), neg_inf, cur)

        @pl.when(has)
        def _():
            out_ref[pl.ds(t, 1), 0:1] = jnp.full((1, 1), mval, f32)
            out_ref[pl.ds(t, 1), 1:2] = jnp.full((1, 1), x1a, f32)
            out_ref[pl.ds(t, 1), 2:3] = jnp.full((1, 1), y1a, f32)
            out_ref[pl.ds(t, 1), 3:4] = jnp.full((1, 1), x2a, f32)
            out_ref[pl.ds(t, 1), 4:5] = jnp.full((1, 1), y2a, f32)

        @pl.when(spec)
        def _():
            out_ref[pl.ds(t + 1, 1), 0:1] = jnp.full((1, 1), m2, f32)
            out_ref[pl.ds(t + 1, 1), 1:2] = jnp.full((1, 1), x1b, f32)
            out_ref[pl.ds(t + 1, 1), 2:3] = jnp.full((1, 1), y1b, f32)
            out_ref[pl.ds(t + 1, 1), 3:4] = jnp.full((1, 1), x2b, f32)
            out_ref[pl.ds(t + 1, 1), 4:5] = jnp.full((1, 1), y2b, f32)

        t = t + jnp.where(spec, jnp.int32(2), jnp.int32(1))
        return t, has, cur

    lax.while_loop(cond, body, (jnp.int32(0), jnp.bool_(True), cur0))


def _sc_compact_kernel(sc_hbm, c0, c1, c2, c3, c4, c5, c6, c7,
                       sco, idxo, o0, o1, o2, o3, o4, o5, o6, o7, cnto,
                       schunk, scv, idxv, idxva, idxvb, gbufs,
                       cntbuf, sem, sem2):
    i32 = jnp.int32
    w = lax.axis_index("s") * 2 + lax.axis_index("c")
    base = w * CHUNK
    pltpu.sync_copy(sc_hbm.at[pl.ds(base, CHUNK)], schunk)

    ninf = jnp.full((16,), -jnp.inf, jnp.float32)
    pad16 = jnp.full((16,), NPAD - 1, i32)
    for j in range(BUF // 16):
        scv[pl.ds(j * 16, 16)] = ninf
        idxv[pl.ds(j * 16, 16)] = pad16

    iota = lax.broadcasted_iota(i32, (16,), 0)
    ptr = i32(0)
    tot = i32(0)
    for j in range(CHUNK // 16):
        v = schunk[pl.ds(j * 16, 16)]
        m = v > CONF_THRESH
        p = jnp.where(m, i32(1), i32(0))
        for d in (1, 2, 4, 8):
            g = p.at[jnp.maximum(iota - d, 0)].get(mode="promise_in_bounds")
            p = p + jnp.where(iota >= d, g, 0)
        sel = jnp.zeros((16,), i32)
        for s in (8, 4, 2, 1):
            nxt = sel + s
            c = p.at[jnp.minimum(nxt - 1, 15)].get(mode="promise_in_bounds")
            sel = jnp.where(c < iota + 1, nxt, sel)
        selc = jnp.minimum(sel, 15)
        scv[pl.ds(ptr, 16)] = v.at[selc].get(mode="promise_in_bounds")
        idxv[pl.ds(ptr, 16)] = base + j * 16 + selc
        cnt = p[15]
        ptr = jnp.minimum(ptr + cnt, i32(CAP))
        tot = tot + cnt
    scv[pl.ds(ptr, 16)] = ninf
    idxv[pl.ds(ptr, 16)] = pad16

    for j in range(128 // 16):
        idxva[pl.ds(j * 16, 16)] = idxv[pl.ds(j * 16, 16)]
        idxvb[pl.ds(j * 16, 16)] = idxv[pl.ds(128 + j * 16, 16)]

    cols8 = (c0, c1, c2, c3, c4, c5, c6, c7)
    descs = []
    for k, col in enumerate(cols8):
        descs.append(pltpu.async_copy(col.at[idxva], gbufs.at[2 * k], sem))
        descs.append(pltpu.async_copy(col.at[idxvb], gbufs.at[2 * k + 1], sem))

    cntbuf[...] = jnp.full((16,), tot, i32)
    wdescs = [pltpu.async_copy(cntbuf, cnto.at[w], sem2),
              pltpu.async_copy(scv.at[pl.ds(0, 128)], sco.at[2 * w], sem2),
              pltpu.async_copy(scv.at[pl.ds(128, 128)], sco.at[2 * w + 1], sem2),
              pltpu.async_copy(idxv.at[pl.ds(0, 128)], idxo.at[2 * w], sem2),
              pltpu.async_copy(idxv.at[pl.ds(128, 128)], idxo.at[2 * w + 1], sem2)]

    outs8 = (o0, o1, o2, o3, o4, o5, o6, o7)
    for k in range(8):
        descs[2 * k].wait()
        wdescs.append(pltpu.async_copy(gbufs.at[2 * k], outs8[k].at[2 * w], sem2))
        descs[2 * k + 1].wait()
        wdescs.append(
            pltpu.async_copy(gbufs.at[2 * k + 1], outs8[k].at[2 * w + 1], sem2))
    for d in wdescs:
        d.wait()


def _build_sc_compact():
    return functools.partial(
        pl.kernel,
        mesh=plsc.VectorSubcoreMesh(core_axis_name="c", subcore_axis_name="s"),
        out_type=[
            jax.ShapeDtypeStruct((CROWS, COLS), jnp.float32),
            jax.ShapeDtypeStruct((CROWS, COLS), jnp.int32),
        ] + [jax.ShapeDtypeStruct((CROWS, COLS), jnp.float32)] * 8
          + [jax.ShapeDtypeStruct((NW, 16), jnp.int32)],
        scratch_types=[
            pltpu.VMEM((CHUNK,), jnp.float32),
            pltpu.VMEM((BUF,), jnp.float32),
            pltpu.VMEM((BUF,), jnp.int32),
            pltpu.VMEM((128,), jnp.int32),
            pltpu.VMEM((128,), jnp.int32),
            pltpu.VMEM((16, 128), jnp.float32),
            pltpu.VMEM((16,), jnp.int32),
            pltpu.SemaphoreType.DMA,
            pltpu.SemaphoreType.DMA,
        ],
    )(_sc_compact_kernel)


def _pad_flat(x, fill=0.0):
    return jnp.pad(x, (0, NPAD - N), constant_values=fill)


def kernel(loc_data, conf_data, prior_data):
    loc = jnp.asarray(loc_data).reshape(N, 4)
    conf = jnp.asarray(conf_data)
    priors = jnp.asarray(prior_data)
    cols = [
        _pad_flat(loc[:, 0]), _pad_flat(loc[:, 1]),
        _pad_flat(loc[:, 2]), _pad_flat(loc[:, 3]),
        _pad_flat(priors[:, 0]), _pad_flat(priors[:, 1]),
        _pad_flat(priors[:, 2]), _pad_flat(priors[:, 3]),
    ]
    sc_flat = _pad_flat(conf[:, 1])

    res = _build_sc_compact()(sc_flat, *cols)
    sco, idxo = res[0], res[1]
    gcols = res[2:10]
    counts = res[10]
    overflow = jnp.any(counts[:, 0] > CAP)

    def compact_path(_):
        args = [sco, idxo, *gcols]
        return pl.pallas_call(
            _nms_compact_kernel,
            out_shape=jax.ShapeDtypeStruct((TOP_K, 5), jnp.float32),
        )(*args)

    def full_path(_):
        args = [c.reshape(FROWS, COLS) for c in cols]
        args.append(sc_flat.reshape(FROWS, COLS))
        return pl.pallas_call(
            _nms_full_kernel,
            out_shape=jax.ShapeDtypeStruct((TOP_K, 5), jnp.float32),
            scratch_shapes=[pltpu.VMEM((FROWS, COLS), jnp.float32)] * 6,
        )(*args)

    rows = lax.cond(overflow, full_path, compact_path, None)
    out = jnp.zeros((1, NUM_CLASSES, TOP_K, 5), jnp.float32)
    return out.at[0, 1].set(rows)


# speedup vs baseline: 1.2296x; 1.2296x over previous
import functools

import jax
import jax.numpy as jnp
from jax import lax
from jax.experimental import pallas as pl
from jax.experimental.pallas import tpu as pltpu
from jax.experimental.pallas import tpu_sc as plsc

N = 20000
NPAD = 20480
FROWS = 160
COLS = 128
NUM_CLASSES = 2
TOP_K = 750
CONF_THRESH = 0.75
NMS_THRESH = 0.3
NMS_TOP_K = 5000
VAR0 = 0.1
VAR1 = 0.2
ONE_BITS = 0x3F800000

NW = 32
CHUNK = NPAD // NW
CAP = 240
BUF = 256
CROWS = NW * BUF // COLS


def _nms_core(scores, idxm, lxv, lyv, lwv, lhv, pcxv, pcyv, pwv, phv,
              out_ref, cur_ref, x1r, y1r, x2r, y2r, ar):
    f32 = jnp.float32
    neg_inf = f32(-jnp.inf)

    cx = pcxv + lxv * VAR0 * pwv
    cy = pcyv + lyv * VAR0 * phv
    w = pwv * jnp.exp(lwv * VAR1)
    h = phv * jnp.exp(lhv * VAR1)
    x1 = cx - w / 2
    y1 = cy - h / 2
    x2 = x1 + w
    y2 = y1 + h
    x1r[...] = x1
    y1r[...] = y1
    x2r[...] = x2
    y2r[...] = y2
    ar[...] = (x2 - x1) * (y2 - y1)

    valid = scores > CONF_THRESH
    bits = jax.lax.bitcast_convert_type(scores, jnp.int32)

    def bs_body(_, carry):
        lo, hi = carry
        mid = lo + (hi - lo + 1) // 2
        cnt = jnp.sum(jnp.where(valid & (bits >= mid), 1, 0))
        take = cnt >= NMS_TOP_K
        return jnp.where(take, mid, lo), jnp.where(take, hi, mid - 1)

    thr, _ = lax.fori_loop(0, 32, bs_body, (jnp.int32(0), jnp.int32(ONE_BITS)))

    c_gt = jnp.sum(jnp.where(valid & (bits > thr), 1, 0))
    m_need = NMS_TOP_K - c_gt
    tie = valid & (bits == thr)

    def bs2_body(_, carry):
        lo, hi = carry
        mid = lo + (hi - lo + 1) // 2
        cnt = jnp.sum(jnp.where(tie & (idxm >= mid), 1, 0))
        take = cnt >= m_need
        return jnp.where(take, mid, lo), jnp.where(take, hi, mid - 1)

    cstar, _ = lax.fori_loop(0, 16, bs2_body, (jnp.int32(0), jnp.int32(NPAD)))

    active = valid & ((bits > thr) | (tie & (idxm >= cstar)))
    cur_ref[...] = jnp.where(active, scores, neg_inf)
    out_ref[...] = jnp.zeros((TOP_K, 5), f32)

    def cond(carry):
        t, alive = carry
        return (t < TOP_K) & alive

    def body(carry):
        t, _ = carry
        cur = cur_ref[...]
        mval = jnp.max(cur)
        has = mval > neg_inf

        @pl.when(has)
        def _():
            i = jnp.max(jnp.where(cur == mval, idxm, -1))
            sel = idxm == i
            x1v = x1r[...]
            y1v = y1r[...]
            x2v = x2r[...]
            y2v = y2r[...]
            av = ar[...]
            zero = f32(0.0)
            x1i = jnp.sum(jnp.where(sel, x1v, zero))
            y1i = jnp.sum(jnp.where(sel, y1v, zero))
            x2i = jnp.sum(jnp.where(sel, x2v, zero))
            y2i = jnp.sum(jnp.where(sel, y2v, zero))
            ai = jnp.sum(jnp.where(sel, av, zero))
            xx1 = jnp.maximum(x1v, x1i)
            yy1 = jnp.maximum(y1v, y1i)
            xx2 = jnp.minimum(x2v, x2i)
            yy2 = jnp.minimum(y2v, y2i)
            iw = jnp.maximum(xx2 - xx1, zero)
            ih = jnp.maximum(yy2 - yy1, zero)
            inter = iw * ih
            union = av - inter + ai
            iou = inter / union
            supp = (iou > NMS_THRESH) | sel
            cur_ref[...] = jnp.where(supp, neg_inf, cur)
            out_ref[pl.ds(t, 1), 0:1] = jnp.full((1, 1), mval, f32)
            out_ref[pl.ds(t, 1), 1:2] = jnp.full((1, 1), x1i, f32)
            out_ref[pl.ds(t, 1), 2:3] = jnp.full((1, 1), y1i, f32)
            out_ref[pl.ds(t, 1), 3:4] = jnp.full((1, 1), x2i, f32)
            out_ref[pl.ds(t, 1), 4:5] = jnp.full((1, 1), y2i, f32)

        return t + 1, has

    lax.while_loop(cond, body, (jnp.int32(0), jnp.bool_(True)))


def _nms_full_kernel(lx, ly, lw, lh, pcx, pcy, pw, ph, sc,
                     out_ref, cur_ref, x1r, y1r, x2r, y2r, ar):
    ridx = lax.broadcasted_iota(jnp.int32, (FROWS, COLS), 0)
    cidx = lax.broadcasted_iota(jnp.int32, (FROWS, COLS), 1)
    idxm = ridx * COLS + cidx
    _nms_core(sc[...], idxm, lx[...], ly[...], lw[...], lh[...],
              pcx[...], pcy[...], pw[...], ph[...],
              out_ref, cur_ref, x1r, y1r, x2r, y2r, ar)


def _nms_compact_kernel(sc, idx, lx, ly, lw, lh, pcx, pcy, pw, ph, out_ref):
    f32 = jnp.float32
    neg_inf = f32(-jnp.inf)
    scores = sc[...]
    idxm = idx[...]

    cx = pcx[...] + lx[...] * VAR0 * pw[...]
    cy = pcy[...] + ly[...] * VAR0 * ph[...]
    w = pw[...] * jnp.exp(lw[...] * VAR1)
    h = ph[...] * jnp.exp(lh[...] * VAR1)
    x1 = cx - w / 2
    y1 = cy - h / 2
    x2 = x1 + w
    y2 = y1 + h
    area = (x2 - x1) * (y2 - y1)

    valid = scores > CONF_THRESH
    bits = jax.lax.bitcast_convert_type(scores, jnp.int32)

    def bs_body(_, carry):
        lo, hi = carry
        mid = lo + (hi - lo + 1) // 2
        cnt = jnp.sum(jnp.where(valid & (bits >= mid), 1, 0))
        take = cnt >= NMS_TOP_K
        return jnp.where(take, mid, lo), jnp.where(take, hi, mid - 1)

    thr, _ = lax.fori_loop(0, 32, bs_body, (jnp.int32(0), jnp.int32(ONE_BITS)))
    c_gt = jnp.sum(jnp.where(valid & (bits > thr), 1, 0))
    m_need = NMS_TOP_K - c_gt
    tie = valid & (bits == thr)

    def bs2_body(_, carry):
        lo, hi = carry
        mid = lo + (hi - lo + 1) // 2
        cnt = jnp.sum(jnp.where(tie & (idxm >= mid), 1, 0))
        take = cnt >= m_need
        return jnp.where(take, mid, lo), jnp.where(take, hi, mid - 1)

    cstar, _ = lax.fori_loop(0, 16, bs2_body, (jnp.int32(0), jnp.int32(NPAD)))

    active = valid & ((bits > thr) | (tie & (idxm >= cstar)))
    cur0 = jnp.where(active, scores, neg_inf)
    out_ref[...] = jnp.zeros((TOP_K, 5), f32)

    def cond(carry):
        t, alive, _ = carry
        return (t < TOP_K) & alive

    def body(carry):
        t, _, cur = carry
        zero = f32(0.0)
        mval = jnp.max(cur)
        has = mval > neg_inf
        sel1m = cur == mval
        i1 = jnp.max(jnp.where(sel1m, idxm, -1))
        n1 = jnp.sum(jnp.where(sel1m, 1, 0))
        cur2 = jnp.where(sel1m, neg_inf, cur)
        m2 = jnp.max(cur2)
        has2 = m2 > neg_inf
        sel2m = cur2 == m2
        i2 = jnp.max(jnp.where(sel2m, idxm, -1))
        n2 = jnp.sum(jnp.where(sel2m, 1, 0))

        sel1 = idxm == i1
        x1a = jnp.sum(jnp.where(sel1, x1, zero))
        y1a = jnp.sum(jnp.where(sel1, y1, zero))
        x2a = jnp.sum(jnp.where(sel1, x2, zero))
        y2a = jnp.sum(jnp.where(sel1, y2, zero))
        aa = (x2a - x1a) * (y2a - y1a)
        iwa = jnp.maximum(jnp.minimum(x2, x2a) - jnp.maximum(x1, x1a), zero)
        iha = jnp.maximum(jnp.minimum(y2, y2a) - jnp.maximum(y1, y1a), zero)
        intera = iwa * iha
        ioua = intera / (area - intera + aa)

        sel2 = idxm == i2
        x1b = jnp.sum(jnp.where(sel2, x1, zero))
        y1b = jnp.sum(jnp.where(sel2, y1, zero))
        x2b = jnp.sum(jnp.where(sel2, x2, zero))
        y2b = jnp.sum(jnp.where(sel2, y2, zero))
        ab = (x2b - x1b) * (y2b - y1b)
        iwb = jnp.maximum(jnp.minimum(x2, x2b) - jnp.maximum(x1, x1b), zero)
        ihb = jnp.maximum(jnp.minimum(y2, y2b) - jnp.maximum(y1, y1b), zero)
        interb = iwb * ihb
        ioub = interb / (area - interb + ab)

        iw12 = jnp.maximum(jnp.minimum(x2b, x2a) - jnp.maximum(x1b, x1a), zero)
        ih12 = jnp.maximum(jnp.minimum(y2b, y2a) - jnp.maximum(y1b, y1a), zero)
        inter12 = iw12 * ih12
        iou12 = inter12 / (ab - inter12 + aa)

        spec = (has2 & (n1 == 1) & (n2 == 1)
                & (iou12 <= NMS_THRESH) & (t < TOP_K - 1))

        cur = jnp.where(has & (ioua > NMS_THRESH), neg_inf, cur)
        cur = jnp.where(spec & (ioub > NMS_THRESH), neg_inf, cur)

        @pl.when(has)
        def _():
            out_ref[pl.ds(t, 1), 0:1] = jnp.full((1, 1), mval, f32)
            out_ref[pl.ds(t, 1), 1:2] = jnp.full((1, 1), x1a, f32)
            out_ref[pl.ds(t, 1), 2:3] = jnp.full((1, 1), y1a, f32)
            out_ref[pl.ds(t, 1), 3:4] = jnp.full((1, 1), x2a, f32)
            out_ref[pl.ds(t, 1), 4:5] = jnp.full((1, 1), y2a, f32)

        @pl.when(spec)
        def _():
            out_ref[pl.ds(t + 1, 1), 0:1] = jnp.full((1, 1), m2, f32)
            out_ref[pl.ds(t + 1, 1), 1:2] = jnp.full((1, 1), x1b, f32)
            out_ref[pl.ds(t + 1, 1), 2:3] = jnp.full((1, 1), y1b, f32)
            out_ref[pl.ds(t + 1, 1), 3:4] = jnp.full((1, 1), x2b, f32)
            out_ref[pl.ds(t + 1, 1), 4:5] = jnp.full((1, 1), y2b, f32)

        t = t + jnp.where(spec, jnp.int32(2), jnp.int32(1))
        return t, has, cur

    lax.while_loop(cond, body, (jnp.int32(0), jnp.bool_(True), cur0))


def _sc_compact_kernel(sc_hbm, c0, c1, c2, c3, c4, c5, c6, c7,
                       sco, idxo, o0, o1, o2, o3, o4, o5, o6, o7, cnto,
                       schunk, scv, idxv, idxva, idxvb, gbufs,
                       cntbuf, sem, sem2):
    i32 = jnp.int32
    w = lax.axis_index("s") * 2 + lax.axis_index("c")
    base = w * CHUNK
    pltpu.sync_copy(sc_hbm.at[pl.ds(base, CHUNK)], schunk)

    ninf = jnp.full((16,), -jnp.inf, jnp.float32)
    pad16 = jnp.full((16,), NPAD - 1, i32)
    for j in range(BUF // 16):
        scv[pl.ds(j * 16, 16)] = ninf
        idxv[pl.ds(j * 16, 16)] = pad16

    iota = lax.broadcasted_iota(i32, (16,), 0)
    ptr = i32(0)
    tot = i32(0)
    for j in range(CHUNK // 16):
        v = schunk[pl.ds(j * 16, 16)]
        m = v > CONF_THRESH
        p = jnp.where(m, i32(1), i32(0))
        for d in (1, 2, 4, 8):
            g = p.at[jnp.maximum(iota - d, 0)].get(mode="promise_in_bounds")
            p = p + jnp.where(iota >= d, g, 0)
        sel = jnp.zeros((16,), i32)
        for s in (8, 4, 2, 1):
            nxt = sel + s
            c = p.at[jnp.minimum(nxt - 1, 15)].get(mode="promise_in_bounds")
            sel = jnp.where(c < iota + 1, nxt, sel)
        selc = jnp.minimum(sel, 15)
        scv[pl.ds(ptr, 16)] = v.at[selc].get(mode="promise_in_bounds")
        idxv[pl.ds(ptr, 16)] = base + j * 16 + selc
        cnt = p[15]
        ptr = jnp.minimum(ptr + cnt, i32(CAP))
        tot = tot + cnt
    scv[pl.ds(ptr, 16)] = ninf
    idxv[pl.ds(ptr, 16)] = pad16

    for j in range(128 // 16):
        idxva[pl.ds(j * 16, 16)] = idxv[pl.ds(j * 16, 16)]
        idxvb[pl.ds(j * 16, 16)] = idxv[pl.ds(128 + j * 16, 16)]

    cols8 = (c0, c1, c2, c3, c4, c5, c6, c7)
    descs = []
    for k, col in enumerate(cols8):
        descs.append(pltpu.async_copy(col.at[idxva], gbufs.at[2 * k], sem))
        descs.append(pltpu.async_copy(col.at[idxvb], gbufs.at[2 * k + 1], sem))

    cntbuf[...] = jnp.full((16,), tot, i32)
    wdescs = [pltpu.async_copy(cntbuf, cnto.at[w], sem2),
              pltpu.async_copy(scv.at[pl.ds(0, 128)], sco.at[2 * w], sem2),
              pltpu.async_copy(scv.at[pl.ds(128, 128)], sco.at[2 * w + 1], sem2),
              pltpu.async_copy(idxv.at[pl.ds(0, 128)], idxo.at[2 * w], sem2),
              pltpu.async_copy(idxv.at[pl.ds(128, 128)], idxo.at[2 * w + 1], sem2)]

    outs8 = (o0, o1, o2, o3, o4, o5, o6, o7)
    for k in range(8):
        descs[2 * k].wait()
        wdescs.append(pltpu.async_copy(gbufs.at[2 * k], outs8[k].at[2 * w], sem2))
        descs[2 * k + 1].wait()
        wdescs.append(
            pltpu.async_copy(gbufs.at[2 * k + 1], outs8[k].at[2 * w + 1], sem2))
    for d in wdescs:
        d.wait()


def _build_sc_compact():
    return functools.partial(
        pl.kernel,
        mesh=plsc.VectorSubcoreMesh(core_axis_name="c", subcore_axis_name="s"),
        out_type=[
            jax.ShapeDtypeStruct((CROWS, COLS), jnp.float32),
            jax.ShapeDtypeStruct((CROWS, COLS), jnp.int32),
        ] + [jax.ShapeDtypeStruct((CROWS, COLS), jnp.float32)] * 8
          + [jax.ShapeDtypeStruct((NW, 16), jnp.int32)],
        scratch_types=[
            pltpu.VMEM((CHUNK,), jnp.float32),
            pltpu.VMEM((BUF,), jnp.float32),
            pltpu.VMEM((BUF,), jnp.int32),
            pltpu.VMEM((128,), jnp.int32),
            pltpu.VMEM((128,), jnp.int32),
            pltpu.VMEM((16, 128), jnp.float32),
            pltpu.VMEM((16,), jnp.int32),
            pltpu.SemaphoreType.DMA,
            pltpu.SemaphoreType.DMA,
        ],
    )(_sc_compact_kernel)


def _pad_flat(x, fill=0.0):
    return jnp.pad(x, (0, NPAD - N), constant_values=fill)


def kernel(loc_data, conf_data, prior_data):
    loc = jnp.asarray(loc_data).reshape(N, 4)
    conf = jnp.asarray(conf_data)
    priors = jnp.asarray(prior_data)
    cols = [
        _pad_flat(loc[:, 0]), _pad_flat(loc[:, 1]),
        _pad_flat(loc[:, 2]), _pad_flat(loc[:, 3]),
        _pad_flat(priors[:, 0]), _pad_flat(priors[:, 1]),
        _pad_flat(priors[:, 2]), _pad_flat(priors[:, 3]),
    ]
    sc_flat = _pad_flat(conf[:, 1])

    res = _build_sc_compact()(sc_flat, *cols)
    sco, idxo = res[0], res[1]
    gcols = res[2:10]
    counts = res[10]
    overflow = jnp.any(counts[:, 0] > CAP)

    def compact_path(_):
        args = [sco, idxo, *gcols]
        return pl.pallas_call(
            _nms_compact_kernel,
            out_shape=jax.ShapeDtypeStruct((TOP_K, 5), jnp.float32),
        )(*args)

    def full_path(_):
        args = [c.reshape(FROWS, COLS) for c in cols]
        args.append(sc_flat.reshape(FROWS, COLS))
        return pl.pallas_call(
            _nms_full_kernel,
            out_shape=jax.ShapeDtypeStruct((TOP_K, 5), jnp.float32),
            scratch_shapes=[pltpu.VMEM((FROWS, COLS), jnp.float32)] * 6,
        )(*args)

    rows = lax.cond(overflow, full_path, compact_path, None)
    out = jnp.zeros((1, NUM_CLASSES, TOP_K, 5), jnp.float32)
    return out.at[0, 1].set(rows)
